# baseline (speedup 1.0000x reference)
import dataclasses
import functools

import jax
import jax.numpy as jnp
from jax import lax
from jax.experimental import pallas as pl
from jax.experimental.pallas import tpu as pltpu
from jax.experimental.pallas import tpu_sc as plsc

N = 10000
E = 320000
D = 128

NB = 10
BR = N // NB
EROWS = 2500
EBLK = 250
ENB = EROWS // EBLK

NC = 2
NS = 16
NW = NC * NS
EPW = E // NW
CH = 80
EPT = E // NS
NCH = EPT // CH
NHALF = 5120
NPSH = NHALF // NS
NPAD = NC * NHALF
MPAD = N + 8

_DN = (((1,), (0,)), ((), ()))


def _dot(a, b):
    return lax.dot_general(a, b, _DN, preferred_element_type=jnp.float32,
                           precision=lax.Precision.DEFAULT)


def _dense1_body(inp_ref, h_ref, we_ref, be_ref, w1_ref, w2_ref,
                 bp_ref, m_ref, q_ref):
    z = jnp.maximum(_dot(inp_ref[...], we_ref[...]) + be_ref[...], 0.0)
    pin = jnp.concatenate([z, h_ref[...]], axis=1)
    m_ref[...] = _dot(pin, w1_ref[...])
    q_ref[...] = _dot(pin, w2_ref[...]) + bp_ref[...]


_dense1 = pl.pallas_call(
    _dense1_body,
    grid=(NB,),
    in_specs=[
        pl.BlockSpec((BR, 2), lambda i: (i, 0)),
        pl.BlockSpec((BR, D), lambda i: (i, 0)),
        pl.BlockSpec((2, D), lambda i: (0, 0)),
        pl.BlockSpec((1, D), lambda i: (0, 0)),
        pl.BlockSpec((2 * D, D), lambda i: (0, 0)),
        pl.BlockSpec((2 * D, D), lambda i: (0, 0)),
        pl.BlockSpec((1, D), lambda i: (0, 0)),
    ],
    out_specs=[pl.BlockSpec((BR, D), lambda i: (i, 0)),
               pl.BlockSpec((BR, D), lambda i: (i, 0))],
    out_shape=[jax.ShapeDtypeStruct((N, D), jnp.float32),
               jax.ShapeDtypeStruct((N, D), jnp.float32)],
)


def _dense2_body(q_ref, a0_ref, wdu_ref, wdv_ref, h_ref, u_ref, v_ref):
    hh = jnp.maximum(q_ref[...] + a0_ref[...], 0.0)
    h_ref[...] = hh
    u_ref[...] = _dot(hh, wdu_ref[...])
    v_ref[...] = _dot(hh, wdv_ref[...])


_dense2 = pl.pallas_call(
    _dense2_body,
    grid=(NB,),
    in_specs=[
        pl.BlockSpec((BR, D), lambda i: (i, 0)),
        pl.BlockSpec((BR, D), lambda i: (i, 0)),
        pl.BlockSpec((D, 1), lambda i: (0, 0)),
        pl.BlockSpec((D, 1), lambda i: (0, 0)),
    ],
    out_specs=[pl.BlockSpec((BR, D), lambda i: (i, 0)),
               pl.BlockSpec((BR, 1), lambda i: (i, 0)),
               pl.BlockSpec((BR, 1), lambda i: (i, 0))],
    out_shape=[jax.ShapeDtypeStruct((N, D), jnp.float32),
               jax.ShapeDtypeStruct((N, 1), jnp.float32),
               jax.ShapeDtypeStruct((N, 1), jnp.float32)],
)


def _edge_bce_body(lg_ref, t_ref, o_ref):
    x = lg_ref[...]
    p = 1.0 / (1.0 + jnp.exp(-x))
    p = jnp.clip(p, 1e-7, 1.0 - 1e-7)
    t = t_ref[...]
    s = jnp.sum(t * jnp.log(p) + (1.0 - t) * jnp.log(1.0 - p))
    o_ref[...] = s.reshape(1, 1)


_edge_bce = pl.pallas_call(
    _edge_bce_body,
    grid=(1,),
    in_specs=[pl.BlockSpec((EROWS, D), lambda i: (0, 0)),
              pl.BlockSpec((EROWS, D), lambda i: (0, 0))],
    out_specs=pl.BlockSpec((1, 1), lambda i: (0, 0)),
    out_shape=jax.ShapeDtypeStruct((1, 1), jnp.float32),
)


def _ynode_body(c_ref, t_ref, y_ref, o_ref):
    cnt = jnp.sum(c_ref[...], axis=0)
    yv = jnp.where(cnt > 0.0, 1.0, 0.0)
    y_ref[...] = yv.reshape(1, N)
    p = jnp.clip(yv, 1e-7, 1.0 - 1e-7)
    t = t_ref[...].reshape(N)
    s = jnp.sum(t * jnp.log(p) + (1.0 - t) * jnp.log(1.0 - p))
    o_ref[...] = s.reshape(1, 1)


_ynode = pl.pallas_call(
    _ynode_body,
    grid=(1,),
    in_specs=[pl.BlockSpec((NW, N), lambda i: (0, 0)),
              pl.BlockSpec((1, N), lambda i: (0, 0))],
    out_specs=[pl.BlockSpec((1, N), lambda i: (0, 0)),
               pl.BlockSpec((1, 1), lambda i: (0, 0))],
    out_shape=[jax.ShapeDtypeStruct((1, N), jnp.float32),
               jax.ShapeDtypeStruct((1, 1), jnp.float32)],
)


_MESH = plsc.VectorSubcoreMesh(core_axis_name="c", subcore_axis_name="s")

_SC_PARAMS = pltpu.CompilerParams()
if "needs_layout_passes" in pltpu.CompilerParams.__dataclass_fields__:
    _SC_PARAMS = dataclasses.replace(_SC_PARAMS, needs_layout_passes=False)


RING = 2


@functools.partial(
    pl.kernel,
    out_type=jax.ShapeDtypeStruct((NC, NHALF, D), jnp.float32),
    mesh=_MESH,
    scratch_types=[
        pltpu.VMEM((CH,), jnp.int32),
        pltpu.VMEM((CH,), jnp.int32),
        pltpu.VMEM((CH, D), jnp.float32),
        pltpu.VMEM((NPSH // 5, D), jnp.float32),
        pltpu.VMEM_SHARED((NHALF, D), jnp.float32),
        pltpu.SemaphoreType.DMA,
    ],
    compiler_params=_SC_PARAMS,
)
def _sc_agg(m_hbm, src_hbm, dst_hbm, out_hbm, sidx, didx, rows, zbuf, acc,
            sem):
    core = lax.axis_index("c")
    sid = lax.axis_index("s")
    nbase = core * NHALF

    zv = jnp.zeros((16,), jnp.float32)

    @pl.loop(0, NPSH // 5)
    def _(r):
        for jj in range(D // 16):
            zbuf[r, pl.ds(jj * 16, 16)] = zv

    @pl.loop(0, 5)
    def _(k):
        pltpu.sync_copy(zbuf, acc.at[pl.ds(sid * NPSH + k * (NPSH // 5),
                                           NPSH // 5)])

    plsc.subcore_barrier()

    @pl.loop(0, NCH)
    def _(ci):
        base = sid * EPT + ci * CH
        pltpu.sync_copy(src_hbm.at[pl.ds(base, CH)], sidx)
        pltpu.sync_copy(dst_hbm.at[pl.ds(base, CH)], didx)
        for jj in range(CH // 16):
            sl = pl.ds(jj * 16, 16)
            d = didx[sl] - nbase
            ok = (d >= 0) & (d < NHALF)
            didx[sl] = jnp.where(ok, d, 0)
            sidx[sl] = jnp.where(ok, sidx[sl], N)
        pltpu.async_copy(m_hbm.at[sidx], rows, sem).wait()
        pltpu.sync_copy(rows, acc.at[didx], add=True)

    plsc.subcore_barrier()
    pltpu.sync_copy(acc.at[pl.ds(sid * NPSH, NPSH)],
                    out_hbm.at[core].at[pl.ds(sid * NPSH, NPSH)])


@functools.partial(
    pl.kernel,
    out_type=[jax.ShapeDtypeStruct((E,), jnp.float32),
              jax.ShapeDtypeStruct((NW * N,), jnp.float32)],
    mesh=_MESH,
    scratch_types=[
        pltpu.VMEM((N,), jnp.float32),
        pltpu.VMEM((N,), jnp.float32),
        pltpu.VMEM((EPW,), jnp.int32),
        pltpu.VMEM((EPW,), jnp.int32),
        pltpu.VMEM((EPW,), jnp.float32),
        pltpu.VMEM((N,), jnp.float32),
        pltpu.VMEM((16,), jnp.float32),
        pltpu.VMEM((16,), jnp.float32),
    ],
    compiler_params=_SC_PARAMS,
)
def _sc_edges(u_hbm, v_hbm, src_hbm, dst_hbm, thr_hbm, b_hbm, lg_hbm,
              cnt_hbm, uv, vv, sidx, didx, lgv, cntv, thrv, bvv):
    core = lax.axis_index("c")
    sid = lax.axis_index("s")
    wid = core * NS + sid
    base = wid * EPW

    pltpu.sync_copy(u_hbm, uv)
    pltpu.sync_copy(v_hbm, vv)
    pltpu.sync_copy(src_hbm.at[pl.ds(base, EPW)], sidx)
    pltpu.sync_copy(dst_hbm.at[pl.ds(base, EPW)], didx)
    pltpu.sync_copy(thr_hbm, thrv)
    pltpu.sync_copy(b_hbm, bvv)

    zv = jnp.zeros((16,), jnp.float32)
    ones = jnp.ones((16,), jnp.float32)

    @pl.loop(0, N // 16)
    def _(i):
        cntv[pl.ds(i * 16, 16)] = zv

    thr = thrv[...]
    bv = bvv[...]

    @pl.loop(0, EPW // 16)
    def _(i):
        sl = pl.ds(i * 16, 16)
        si = sidx[sl]
        di = didx[sl]
        uu = plsc.load_gather(uv, [si])
        vg = plsc.load_gather(vv, [di])
        lg = uu + vg + bv
        lgv[sl] = lg
        ind = jnp.where(lg >= thr, ones, zv)
        plsc.addupdate_scatter(cntv, [si], ind)
        plsc.addupdate_scatter(cntv, [di], ind)

    pltpu.sync_copy(lgv, lg_hbm.at[pl.ds(base, EPW)])
    pltpu.sync_copy(cntv, cnt_hbm.at[pl.ds(wid * N, N)])


def _sigmoid_threshold():
    lo = jnp.float32(1.0)
    hi = jnp.float32(2.0)

    def body(_, lh):
        lo_, hi_ = lh
        lob = lax.bitcast_convert_type(lo_, jnp.uint32)
        hib = lax.bitcast_convert_type(hi_, jnp.uint32)
        mid = lax.bitcast_convert_type((lob + hib) // 2, jnp.float32)
        up = jax.nn.sigmoid(mid) >= jnp.float32(0.8)
        return (jnp.where(up, lo_, mid), jnp.where(up, mid, hi_))

    lo, hi = lax.fori_loop(0, 32, body, (lo, hi))
    return hi


def kernel(pos, s, pi, pi_h, reach_h, edge_index, W_enc, b_enc, W1, W2, b_p,
           W_dec, b_dec):
    f32 = jnp.float32
    src = edge_index[0].astype(jnp.int32)
    dst = edge_index[1].astype(jnp.int32)

    pos_c = pos.astype(f32).reshape(N, 1)
    be = b_enc.astype(f32).reshape(1, D)
    bp = b_p.astype(f32).reshape(1, D)
    wdu = W_dec[:D, :].astype(f32).reshape(D, 1)
    wdv = W_dec[D:, :].astype(f32).reshape(D, 1)
    W_enc = W_enc.astype(f32)
    W1 = W1.astype(f32)
    W2 = W2.astype(f32)

    thr = jnp.full((16,), _sigmoid_threshold(), f32)
    bsp = jnp.full((16,), b_dec[0].astype(f32), f32)

    edge_t = [pi_h[i].astype(f32).reshape(EROWS, D) for i in range(1, 5)]
    edge_t.append(pi.astype(f32).reshape(EROWS, D))
    node_t = [reach_h[i].astype(f32).reshape(1, N) for i in range(1, 5)]
    node_t.append(reach_h[4].astype(f32).reshape(1, N))

    h0 = jnp.zeros((N, D), f32)
    ycol0 = s.astype(f32).reshape(N, 1)
    mzpad = jnp.zeros((MPAD - N, D), f32)

    h, ycol = h0, ycol0
    edge_sums = []
    node_sums = []
    for it in range(5):
        inp = jnp.concatenate([pos_c, ycol], axis=1)
        m, q = _dense1(inp, h, W_enc, be, W1, W2, bp)
        aggp = _sc_agg(jnp.concatenate([m, mzpad], axis=0), src, dst)
        h, u, v = _dense2(q, aggp.reshape(NPAD, D)[:N], wdu, wdv)
        lg, cnt = _sc_edges(u.reshape(N), v.reshape(N), src, dst, thr, bsp)
        y_row, ysum = _ynode(cnt.reshape(NW, N), node_t[it])
        esum = _edge_bce(lg.reshape(EROWS, D), edge_t[it])
        edge_sums.append(esum[0, 0])
        node_sums.append(ysum[0, 0])
        ycol = y_row.reshape(N, 1)
    ycol_f = ycol

    loss_x = -(edge_sums[4] / E)
    loss_h = 0.0
    for it in range(4):
        loss_h = loss_h + (-(edge_sums[it] / E))
    yloss_x = -(node_sums[4] / N)
    yloss_h = 0.0
    for it in range(4):
        yloss_h = yloss_h + (-(node_sums[it] / N))

    y = ycol_f.reshape(N)
    return (y, loss_x, loss_h, yloss_x, yloss_h)


# speedup vs baseline: 18.0001x; 18.0001x over previous
import dataclasses
import functools

import jax
import jax.numpy as jnp
from jax import lax
from jax.experimental import pallas as pl
from jax.experimental.pallas import tpu as pltpu
from jax.experimental.pallas import tpu_sc as plsc

N = 10000
E = 320000
D = 128

NB = 10
BR = N // NB
EROWS = 2500
EBLK = 250
ENB = EROWS // EBLK

NC = 2
NS = 16
NW = NC * NS
EPW = E // NW
CH = 80
EPT = E // NS
NCH = EPT // CH
NHALF = 5120
NPSH = NHALF // NS
NPAD = NC * NHALF
NACC = NHALF + 8

_DN = (((1,), (0,)), ((), ()))


def _dot(a, b):
    return lax.dot_general(a, b, _DN, preferred_element_type=jnp.float32,
                           precision=lax.Precision.DEFAULT)


def _dense1_body(inp_ref, h_ref, we_ref, be_ref, w1_ref, w2_ref,
                 bp_ref, m_ref, q_ref):
    z = jnp.maximum(_dot(inp_ref[...], we_ref[...]) + be_ref[...], 0.0)
    pin = jnp.concatenate([z, h_ref[...]], axis=1)
    m_ref[...] = _dot(pin, w1_ref[...])
    q_ref[...] = _dot(pin, w2_ref[...]) + bp_ref[...]


_dense1 = pl.pallas_call(
    _dense1_body,
    grid=(NB,),
    in_specs=[
        pl.BlockSpec((BR, 2), lambda i: (i, 0)),
        pl.BlockSpec((BR, D), lambda i: (i, 0)),
        pl.BlockSpec((2, D), lambda i: (0, 0)),
        pl.BlockSpec((1, D), lambda i: (0, 0)),
        pl.BlockSpec((2 * D, D), lambda i: (0, 0)),
        pl.BlockSpec((2 * D, D), lambda i: (0, 0)),
        pl.BlockSpec((1, D), lambda i: (0, 0)),
    ],
    out_specs=[pl.BlockSpec((BR, D), lambda i: (i, 0)),
               pl.BlockSpec((BR, D), lambda i: (i, 0))],
    out_shape=[jax.ShapeDtypeStruct((N, D), jnp.float32),
               jax.ShapeDtypeStruct((N, D), jnp.float32)],
)


def _dense2_body(q_ref, a0_ref, wdu_ref, wdv_ref, h_ref, u_ref, v_ref):
    hh = jnp.maximum(q_ref[...] + a0_ref[...], 0.0)
    h_ref[...] = hh
    u_ref[...] = _dot(hh, wdu_ref[...])
    v_ref[...] = _dot(hh, wdv_ref[...])


_dense2 = pl.pallas_call(
    _dense2_body,
    grid=(NB,),
    in_specs=[
        pl.BlockSpec((BR, D), lambda i: (i, 0)),
        pl.BlockSpec((BR, D), lambda i: (i, 0)),
        pl.BlockSpec((D, 1), lambda i: (0, 0)),
        pl.BlockSpec((D, 1), lambda i: (0, 0)),
    ],
    out_specs=[pl.BlockSpec((BR, D), lambda i: (i, 0)),
               pl.BlockSpec((BR, 1), lambda i: (i, 0)),
               pl.BlockSpec((BR, 1), lambda i: (i, 0))],
    out_shape=[jax.ShapeDtypeStruct((N, D), jnp.float32),
               jax.ShapeDtypeStruct((N, 1), jnp.float32),
               jax.ShapeDtypeStruct((N, 1), jnp.float32)],
)


def _edge_bce_body(lg_ref, t_ref, o_ref):
    x = lg_ref[...]
    p = 1.0 / (1.0 + jnp.exp(-x))
    p = jnp.clip(p, 1e-7, 1.0 - 1e-7)
    t = t_ref[...]
    s = jnp.sum(t * jnp.log(p) + (1.0 - t) * jnp.log(1.0 - p))
    o_ref[...] = s.reshape(1, 1)


_edge_bce = pl.pallas_call(
    _edge_bce_body,
    grid=(1,),
    in_specs=[pl.BlockSpec((EROWS, D), lambda i: (0, 0)),
              pl.BlockSpec((EROWS, D), lambda i: (0, 0))],
    out_specs=pl.BlockSpec((1, 1), lambda i: (0, 0)),
    out_shape=jax.ShapeDtypeStruct((1, 1), jnp.float32),
)


def _ynode_body(c_ref, t_ref, y_ref, o_ref):
    cnt = jnp.sum(c_ref[...], axis=0)
    yv = jnp.where(cnt > 0.0, 1.0, 0.0)
    y_ref[...] = yv.reshape(1, N)
    p = jnp.clip(yv, 1e-7, 1.0 - 1e-7)
    t = t_ref[...].reshape(N)
    s = jnp.sum(t * jnp.log(p) + (1.0 - t) * jnp.log(1.0 - p))
    o_ref[...] = s.reshape(1, 1)


_ynode = pl.pallas_call(
    _ynode_body,
    grid=(1,),
    in_specs=[pl.BlockSpec((NW, N), lambda i: (0, 0)),
              pl.BlockSpec((1, N), lambda i: (0, 0))],
    out_specs=[pl.BlockSpec((1, N), lambda i: (0, 0)),
               pl.BlockSpec((1, 1), lambda i: (0, 0))],
    out_shape=[jax.ShapeDtypeStruct((1, N), jnp.float32),
               jax.ShapeDtypeStruct((1, 1), jnp.float32)],
)


_MESH = plsc.VectorSubcoreMesh(core_axis_name="c", subcore_axis_name="s")

_SC_PARAMS = pltpu.CompilerParams()
if "needs_layout_passes" in pltpu.CompilerParams.__dataclass_fields__:
    _SC_PARAMS = dataclasses.replace(_SC_PARAMS, needs_layout_passes=False)


RING = 2


@functools.partial(
    pl.kernel,
    out_type=jax.ShapeDtypeStruct((NC, NHALF, D), jnp.float32),
    mesh=_MESH,
    scratch_types=[
        pltpu.VMEM((CH,), jnp.int32),
        pltpu.VMEM((CH,), jnp.int32),
        pltpu.VMEM((CH, D), jnp.float32),
        pltpu.VMEM((NPSH // 5, D), jnp.float32),
        pltpu.VMEM_SHARED((NACC, D), jnp.float32),
        pltpu.SemaphoreType.DMA,
    ],
    compiler_params=_SC_PARAMS,
)
def _sc_agg(m_hbm, src_hbm, dst_hbm, out_hbm, sidx, didx, rows, zbuf, acc,
            sem):
    core = lax.axis_index("c")
    sid = lax.axis_index("s")
    nbase = core * NHALF

    zv = jnp.zeros((16,), jnp.float32)

    @pl.loop(0, NPSH // 5)
    def _(r):
        for jj in range(D // 16):
            zbuf[r, pl.ds(jj * 16, 16)] = zv

    @pl.loop(0, 5)
    def _(k):
        pltpu.sync_copy(zbuf, acc.at[pl.ds(sid * NPSH + k * (NPSH // 5),
                                           NPSH // 5)])

    plsc.subcore_barrier()

    @pl.loop(0, NCH)
    def _(ci):
        base = sid * EPT + ci * CH
        pltpu.sync_copy(src_hbm.at[pl.ds(base, CH)], sidx)
        pltpu.sync_copy(dst_hbm.at[pl.ds(base, CH)], didx)
        for jj in range(CH // 16):
            sl = pl.ds(jj * 16, 16)
            d = didx[sl] - nbase
            ok = (d >= 0) & (d < NHALF)
            didx[sl] = jnp.where(ok, d, NHALF)
        pltpu.async_copy(m_hbm.at[sidx], rows, sem).wait()
        pltpu.sync_copy(rows, acc.at[didx], add=True)

    plsc.subcore_barrier()
    pltpu.sync_copy(acc.at[pl.ds(sid * NPSH, NPSH)],
                    out_hbm.at[core].at[pl.ds(sid * NPSH, NPSH)])


@functools.partial(
    pl.kernel,
    out_type=[jax.ShapeDtypeStruct((E,), jnp.float32),
              jax.ShapeDtypeStruct((NW * N,), jnp.float32)],
    mesh=_MESH,
    scratch_types=[
        pltpu.VMEM((N,), jnp.float32),
        pltpu.VMEM((N,), jnp.float32),
        pltpu.VMEM((EPW,), jnp.int32),
        pltpu.VMEM((EPW,), jnp.int32),
        pltpu.VMEM((EPW,), jnp.float32),
        pltpu.VMEM((N,), jnp.float32),
        pltpu.VMEM((16,), jnp.float32),
        pltpu.VMEM((16,), jnp.float32),
    ],
    compiler_params=_SC_PARAMS,
)
def _sc_edges(u_hbm, v_hbm, src_hbm, dst_hbm, thr_hbm, b_hbm, lg_hbm,
              cnt_hbm, uv, vv, sidx, didx, lgv, cntv, thrv, bvv):
    core = lax.axis_index("c")
    sid = lax.axis_index("s")
    wid = core * NS + sid
    base = wid * EPW

    pltpu.sync_copy(u_hbm, uv)
    pltpu.sync_copy(v_hbm, vv)
    pltpu.sync_copy(src_hbm.at[pl.ds(base, EPW)], sidx)
    pltpu.sync_copy(dst_hbm.at[pl.ds(base, EPW)], didx)
    pltpu.sync_copy(thr_hbm, thrv)
    pltpu.sync_copy(b_hbm, bvv)

    zv = jnp.zeros((16,), jnp.float32)
    ones = jnp.ones((16,), jnp.float32)

    @pl.loop(0, N // 16)
    def _(i):
        cntv[pl.ds(i * 16, 16)] = zv

    thr = thrv[...]
    bv = bvv[...]

    @pl.loop(0, EPW // 16)
    def _(i):
        sl = pl.ds(i * 16, 16)
        si = sidx[sl]
        di = didx[sl]
        uu = plsc.load_gather(uv, [si])
        vg = plsc.load_gather(vv, [di])
        lg = uu + vg + bv
        lgv[sl] = lg
        ind = jnp.where(lg >= thr, ones, zv)
        plsc.addupdate_scatter(cntv, [si], ind)
        plsc.addupdate_scatter(cntv, [di], ind)

    pltpu.sync_copy(lgv, lg_hbm.at[pl.ds(base, EPW)])
    pltpu.sync_copy(cntv, cnt_hbm.at[pl.ds(wid * N, N)])


def _sigmoid_threshold():
    lo = jnp.float32(1.0)
    hi = jnp.float32(2.0)

    def body(_, lh):
        lo_, hi_ = lh
        lob = lax.bitcast_convert_type(lo_, jnp.uint32)
        hib = lax.bitcast_convert_type(hi_, jnp.uint32)
        mid = lax.bitcast_convert_type((lob + hib) // 2, jnp.float32)
        up = jax.nn.sigmoid(mid) >= jnp.float32(0.8)
        return (jnp.where(up, lo_, mid), jnp.where(up, mid, hi_))

    lo, hi = lax.fori_loop(0, 32, body, (lo, hi))
    return hi


def kernel(pos, s, pi, pi_h, reach_h, edge_index, W_enc, b_enc, W1, W2, b_p,
           W_dec, b_dec):
    f32 = jnp.float32
    src = edge_index[0].astype(jnp.int32)
    dst = edge_index[1].astype(jnp.int32)

    pos_c = pos.astype(f32).reshape(N, 1)
    be = b_enc.astype(f32).reshape(1, D)
    bp = b_p.astype(f32).reshape(1, D)
    wdu = W_dec[:D, :].astype(f32).reshape(D, 1)
    wdv = W_dec[D:, :].astype(f32).reshape(D, 1)
    W_enc = W_enc.astype(f32)
    W1 = W1.astype(f32)
    W2 = W2.astype(f32)

    thr = jnp.full((16,), _sigmoid_threshold(), f32)
    bsp = jnp.full((16,), b_dec[0].astype(f32), f32)

    edge_t = [pi_h[i].astype(f32).reshape(EROWS, D) for i in range(1, 5)]
    edge_t.append(pi.astype(f32).reshape(EROWS, D))
    node_t = [reach_h[i].astype(f32).reshape(1, N) for i in range(1, 5)]
    node_t.append(reach_h[4].astype(f32).reshape(1, N))

    h0 = jnp.zeros((N, D), f32)
    ycol0 = s.astype(f32).reshape(N, 1)
    h, ycol = h0, ycol0
    edge_sums = []
    node_sums = []
    for it in range(5):
        inp = jnp.concatenate([pos_c, ycol], axis=1)
        m, q = _dense1(inp, h, W_enc, be, W1, W2, bp)
        aggp = _sc_agg(m, src, dst)
        h, u, v = _dense2(q, aggp.reshape(NPAD, D)[:N], wdu, wdv)
        lg, cnt = _sc_edges(u.reshape(N), v.reshape(N), src, dst, thr, bsp)
        y_row, ysum = _ynode(cnt.reshape(NW, N), node_t[it])
        esum = _edge_bce(lg.reshape(EROWS, D), edge_t[it])
        edge_sums.append(esum[0, 0])
        node_sums.append(ysum[0, 0])
        ycol = y_row.reshape(N, 1)
    ycol_f = ycol

    loss_x = -(edge_sums[4] / E)
    loss_h = 0.0
    for it in range(4):
        loss_h = loss_h + (-(edge_sums[it] / E))
    yloss_x = -(node_sums[4] / N)
    yloss_h = 0.0
    for it in range(4):
        yloss_h = yloss_h + (-(node_sums[it] / N))

    y = ycol_f.reshape(N)
    return (y, loss_x, loss_h, yloss_x, yloss_h)


# speedup vs baseline: 28.0729x; 1.5596x over previous
import dataclasses
import functools

import jax
import jax.numpy as jnp
from jax import lax
from jax.experimental import pallas as pl
from jax.experimental.pallas import tpu as pltpu
from jax.experimental.pallas import tpu_sc as plsc

N = 10000
E = 320000
D = 128

NB = 10
BR = N // NB
EROWS = 2500
EBLK = 250
ENB = EROWS // EBLK

NC = 2
NS = 16
NW = NC * NS
EPW = E // NW
CH = 80
EPT = E // NS
NCH = EPT // CH
NHALF = 5120
NPSH = NHALF // NS
NPAD = NC * NHALF
NACC = NHALF + 8

_DN = (((1,), (0,)), ((), ()))


def _dot(a, b):
    return lax.dot_general(a, b, _DN, preferred_element_type=jnp.float32,
                           precision=lax.Precision.DEFAULT)


def _dense1_body(inp_ref, h_ref, we_ref, be_ref, w1_ref, w2_ref,
                 bp_ref, m_ref, q_ref):
    z = jnp.maximum(_dot(inp_ref[...], we_ref[...]) + be_ref[...], 0.0)
    pin = jnp.concatenate([z, h_ref[...]], axis=1)
    m_ref[...] = _dot(pin, w1_ref[...])
    q_ref[...] = _dot(pin, w2_ref[...]) + bp_ref[...]


_dense1 = pl.pallas_call(
    _dense1_body,
    grid=(NB,),
    in_specs=[
        pl.BlockSpec((BR, 2), lambda i: (i, 0)),
        pl.BlockSpec((BR, D), lambda i: (i, 0)),
        pl.BlockSpec((2, D), lambda i: (0, 0)),
        pl.BlockSpec((1, D), lambda i: (0, 0)),
        pl.BlockSpec((2 * D, D), lambda i: (0, 0)),
        pl.BlockSpec((2 * D, D), lambda i: (0, 0)),
        pl.BlockSpec((1, D), lambda i: (0, 0)),
    ],
    out_specs=[pl.BlockSpec((BR, D), lambda i: (i, 0)),
               pl.BlockSpec((BR, D), lambda i: (i, 0))],
    out_shape=[jax.ShapeDtypeStruct((N, D), jnp.float32),
               jax.ShapeDtypeStruct((N, D), jnp.float32)],
)


def _dense2_body(q_ref, a0_ref, wdu_ref, wdv_ref, h_ref, u_ref, v_ref):
    hh = jnp.maximum(q_ref[...] + a0_ref[...], 0.0)
    h_ref[...] = hh
    u_ref[...] = _dot(hh, wdu_ref[...])
    v_ref[...] = _dot(hh, wdv_ref[...])


_dense2 = pl.pallas_call(
    _dense2_body,
    grid=(NB,),
    in_specs=[
        pl.BlockSpec((BR, D), lambda i: (i, 0)),
        pl.BlockSpec((BR, D), lambda i: (i, 0)),
        pl.BlockSpec((D, 1), lambda i: (0, 0)),
        pl.BlockSpec((D, 1), lambda i: (0, 0)),
    ],
    out_specs=[pl.BlockSpec((BR, D), lambda i: (i, 0)),
               pl.BlockSpec((BR, 1), lambda i: (i, 0)),
               pl.BlockSpec((BR, 1), lambda i: (i, 0))],
    out_shape=[jax.ShapeDtypeStruct((N, D), jnp.float32),
               jax.ShapeDtypeStruct((N, 1), jnp.float32),
               jax.ShapeDtypeStruct((N, 1), jnp.float32)],
)


def _edge_bce_body(lg_ref, t_ref, o_ref):
    x = lg_ref[...]
    p = 1.0 / (1.0 + jnp.exp(-x))
    p = jnp.clip(p, 1e-7, 1.0 - 1e-7)
    t = t_ref[...]
    s = jnp.sum(t * jnp.log(p) + (1.0 - t) * jnp.log(1.0 - p))
    o_ref[...] = s.reshape(1, 1)


_edge_bce = pl.pallas_call(
    _edge_bce_body,
    grid=(1,),
    in_specs=[pl.BlockSpec((EROWS, D), lambda i: (0, 0)),
              pl.BlockSpec((EROWS, D), lambda i: (0, 0))],
    out_specs=pl.BlockSpec((1, 1), lambda i: (0, 0)),
    out_shape=jax.ShapeDtypeStruct((1, 1), jnp.float32),
)


def _ynode_body(c_ref, t_ref, y_ref, o_ref):
    cnt = jnp.sum(c_ref[...], axis=0)
    yv = jnp.where(cnt > 0.0, 1.0, 0.0)
    y_ref[...] = yv.reshape(1, N)
    p = jnp.clip(yv, 1e-7, 1.0 - 1e-7)
    t = t_ref[...].reshape(N)
    s = jnp.sum(t * jnp.log(p) + (1.0 - t) * jnp.log(1.0 - p))
    o_ref[...] = s.reshape(1, 1)


_ynode = pl.pallas_call(
    _ynode_body,
    grid=(1,),
    in_specs=[pl.BlockSpec((NW, N), lambda i: (0, 0)),
              pl.BlockSpec((1, N), lambda i: (0, 0))],
    out_specs=[pl.BlockSpec((1, N), lambda i: (0, 0)),
               pl.BlockSpec((1, 1), lambda i: (0, 0))],
    out_shape=[jax.ShapeDtypeStruct((1, N), jnp.float32),
               jax.ShapeDtypeStruct((1, 1), jnp.float32)],
)


_MESH = plsc.VectorSubcoreMesh(core_axis_name="c", subcore_axis_name="s")

_SC_PARAMS = pltpu.CompilerParams()
if "needs_layout_passes" in pltpu.CompilerParams.__dataclass_fields__:
    _SC_PARAMS = dataclasses.replace(_SC_PARAMS, needs_layout_passes=False)


RING = 2


@functools.partial(
    pl.kernel,
    out_type=jax.ShapeDtypeStruct((NC, NHALF, D), jnp.float32),
    mesh=_MESH,
    scratch_types=[
        pltpu.VMEM((NCH, CH), jnp.int32),
        pltpu.VMEM((NCH, CH), jnp.int32),
        pltpu.VMEM((RING, CH, D), jnp.float32),
        pltpu.VMEM((8, D), jnp.float32),
        pltpu.VMEM_SHARED((NACC, D), jnp.float32),
        pltpu.SemaphoreType.DMA,
        pltpu.SemaphoreType.DMA((RING,)),
    ],
    compiler_params=_SC_PARAMS,
)
def _sc_agg(m_hbm, src_hbm, dst_hbm, out_hbm, sidx, didx, rows, zbuf, acc,
            sem, gsem):
    core = lax.axis_index("c")
    sid = lax.axis_index("s")
    nbase = core * NHALF

    zv = jnp.zeros((16,), jnp.float32)

    @pl.loop(0, 8)
    def _(r):
        for jj in range(D // 16):
            zbuf[r, pl.ds(jj * 16, 16)] = zv

    @pl.loop(0, NPSH // 8)
    def _(k):
        pltpu.sync_copy(zbuf, acc.at[pl.ds(sid * NPSH + k * 8, 8)])

    plsc.subcore_barrier()

    def fetch_remap(ci):
        base = sid * EPT + ci * CH
        pltpu.sync_copy(src_hbm.at[pl.ds(base, CH)], sidx.at[ci])
        pltpu.sync_copy(dst_hbm.at[pl.ds(base, CH)], didx.at[ci])
        for jj in range(CH // 16):
            sl = pl.ds(jj * 16, 16)
            d = didx[ci, sl] - nbase
            ok = (d >= 0) & (d < NHALF)
            didx[ci, sl] = jnp.where(ok, d, NHALF)

    @pl.loop(0, RING)
    def _(r):
        fetch_remap(r)
        pltpu.async_copy(m_hbm.at[sidx.at[r]], rows.at[r], gsem.at[r])

    @pl.loop(0, NCH)
    def _(ci):
        r = lax.rem(ci, RING)
        pltpu.make_async_copy(m_hbm.at[sidx.at[ci]], rows.at[r],
                              gsem.at[r]).wait()
        pltpu.sync_copy(rows.at[r], acc.at[didx.at[ci]], add=True)

        @pl.when(ci + RING < NCH)
        def _():
            fetch_remap(ci + RING)
            pltpu.async_copy(m_hbm.at[sidx.at[ci + RING]], rows.at[r],
                             gsem.at[r])

    plsc.subcore_barrier()
    pltpu.sync_copy(acc.at[pl.ds(sid * NPSH, NPSH)],
                    out_hbm.at[core].at[pl.ds(sid * NPSH, NPSH)])


@functools.partial(
    pl.kernel,
    out_type=[jax.ShapeDtypeStruct((E,), jnp.float32),
              jax.ShapeDtypeStruct((NW * N,), jnp.float32)],
    mesh=_MESH,
    scratch_types=[
        pltpu.VMEM((N,), jnp.float32),
        pltpu.VMEM((N,), jnp.float32),
        pltpu.VMEM((EPW,), jnp.int32),
        pltpu.VMEM((EPW,), jnp.int32),
        pltpu.VMEM((EPW,), jnp.float32),
        pltpu.VMEM((N,), jnp.float32),
        pltpu.VMEM((16,), jnp.float32),
        pltpu.VMEM((16,), jnp.float32),
    ],
    compiler_params=_SC_PARAMS,
)
def _sc_edges(u_hbm, v_hbm, src_hbm, dst_hbm, thr_hbm, b_hbm, lg_hbm,
              cnt_hbm, uv, vv, sidx, didx, lgv, cntv, thrv, bvv):
    core = lax.axis_index("c")
    sid = lax.axis_index("s")
    wid = core * NS + sid
    base = wid * EPW

    pltpu.sync_copy(u_hbm, uv)
    pltpu.sync_copy(v_hbm, vv)
    pltpu.sync_copy(src_hbm.at[pl.ds(base, EPW)], sidx)
    pltpu.sync_copy(dst_hbm.at[pl.ds(base, EPW)], didx)
    pltpu.sync_copy(thr_hbm, thrv)
    pltpu.sync_copy(b_hbm, bvv)

    zv = jnp.zeros((16,), jnp.float32)
    ones = jnp.ones((16,), jnp.float32)

    @pl.loop(0, N // 16)
    def _(i):
        cntv[pl.ds(i * 16, 16)] = zv

    thr = thrv[...]
    bv = bvv[...]

    @pl.loop(0, EPW // 16)
    def _(i):
        sl = pl.ds(i * 16, 16)
        si = sidx[sl]
        di = didx[sl]
        uu = plsc.load_gather(uv, [si])
        vg = plsc.load_gather(vv, [di])
        lg = uu + vg + bv
        lgv[sl] = lg
        ind = jnp.where(lg >= thr, ones, zv)
        plsc.addupdate_scatter(cntv, [si], ind)
        plsc.addupdate_scatter(cntv, [di], ind)

    pltpu.sync_copy(lgv, lg_hbm.at[pl.ds(base, EPW)])
    pltpu.sync_copy(cntv, cnt_hbm.at[pl.ds(wid * N, N)])


def _sigmoid_threshold():
    lo = jnp.float32(1.0)
    hi = jnp.float32(2.0)

    def body(_, lh):
        lo_, hi_ = lh
        lob = lax.bitcast_convert_type(lo_, jnp.uint32)
        hib = lax.bitcast_convert_type(hi_, jnp.uint32)
        mid = lax.bitcast_convert_type((lob + hib) // 2, jnp.float32)
        up = jax.nn.sigmoid(mid) >= jnp.float32(0.8)
        return (jnp.where(up, lo_, mid), jnp.where(up, mid, hi_))

    lo, hi = lax.fori_loop(0, 32, body, (lo, hi))
    return hi


def kernel(pos, s, pi, pi_h, reach_h, edge_index, W_enc, b_enc, W1, W2, b_p,
           W_dec, b_dec):
    f32 = jnp.float32
    src = edge_index[0].astype(jnp.int32)
    dst = edge_index[1].astype(jnp.int32)

    pos_c = pos.astype(f32).reshape(N, 1)
    be = b_enc.astype(f32).reshape(1, D)
    bp = b_p.astype(f32).reshape(1, D)
    wdu = W_dec[:D, :].astype(f32).reshape(D, 1)
    wdv = W_dec[D:, :].astype(f32).reshape(D, 1)
    W_enc = W_enc.astype(f32)
    W1 = W1.astype(f32)
    W2 = W2.astype(f32)

    thr = jnp.full((16,), _sigmoid_threshold(), f32)
    bsp = jnp.full((16,), b_dec[0].astype(f32), f32)

    edge_t = [pi_h[i].astype(f32).reshape(EROWS, D) for i in range(1, 5)]
    edge_t.append(pi.astype(f32).reshape(EROWS, D))
    node_t = [reach_h[i].astype(f32).reshape(1, N) for i in range(1, 5)]
    node_t.append(reach_h[4].astype(f32).reshape(1, N))

    h0 = jnp.zeros((N, D), f32)
    ycol0 = s.astype(f32).reshape(N, 1)
    h, ycol = h0, ycol0
    edge_sums = []
    node_sums = []
    for it in range(5):
        inp = jnp.concatenate([pos_c, ycol], axis=1)
        m, q = _dense1(inp, h, W_enc, be, W1, W2, bp)
        aggp = _sc_agg(m, src, dst)
        h, u, v = _dense2(q, aggp.reshape(NPAD, D)[:N], wdu, wdv)
        lg, cnt = _sc_edges(u.reshape(N), v.reshape(N), src, dst, thr, bsp)
        y_row, ysum = _ynode(cnt.reshape(NW, N), node_t[it])
        esum = _edge_bce(lg.reshape(EROWS, D), edge_t[it])
        edge_sums.append(esum[0, 0])
        node_sums.append(ysum[0, 0])
        ycol = y_row.reshape(N, 1)
    ycol_f = ycol

    loss_x = -(edge_sums[4] / E)
    loss_h = 0.0
    for it in range(4):
        loss_h = loss_h + (-(edge_sums[it] / E))
    yloss_x = -(node_sums[4] / N)
    yloss_h = 0.0
    for it in range(4):
        yloss_h = yloss_h + (-(node_sums[it] / N))

    y = ycol_f.reshape(N)
    return (y, loss_x, loss_h, yloss_x, yloss_h)


# speedup vs baseline: 36.3857x; 1.2961x over previous
import dataclasses
import functools

import jax
import jax.numpy as jnp
from jax import lax
from jax.experimental import pallas as pl
from jax.experimental.pallas import tpu as pltpu
from jax.experimental.pallas import tpu_sc as plsc

N = 10000
E = 320000
D = 128

NB = 10
BR = N // NB
EROWS = 2500
EBLK = 250
ENB = EROWS // EBLK

NC = 2
NS = 16
NW = NC * NS
EPW = E // NW
CH = 80
EPT = E // NS
NCH = EPT // CH
NHALF = 5120
NPSH = NHALF // NS
NPAD = NC * NHALF
NACC = NHALF + 8

_DN = (((1,), (0,)), ((), ()))


def _dot(a, b):
    return lax.dot_general(a, b, _DN, preferred_element_type=jnp.float32,
                           precision=lax.Precision.DEFAULT)


def _dense1_body(inp_ref, h_ref, we_ref, be_ref, w1_ref, w2_ref,
                 bp_ref, m_ref, q_ref):
    z = jnp.maximum(_dot(inp_ref[...], we_ref[...]) + be_ref[...], 0.0)
    pin = jnp.concatenate([z, h_ref[...]], axis=1)
    m_ref[...] = _dot(pin, w1_ref[...])
    q_ref[...] = _dot(pin, w2_ref[...]) + bp_ref[...]


_dense1 = pl.pallas_call(
    _dense1_body,
    grid=(NB,),
    in_specs=[
        pl.BlockSpec((BR, 2), lambda i: (i, 0)),
        pl.BlockSpec((BR, D), lambda i: (i, 0)),
        pl.BlockSpec((2, D), lambda i: (0, 0)),
        pl.BlockSpec((1, D), lambda i: (0, 0)),
        pl.BlockSpec((2 * D, D), lambda i: (0, 0)),
        pl.BlockSpec((2 * D, D), lambda i: (0, 0)),
        pl.BlockSpec((1, D), lambda i: (0, 0)),
    ],
    out_specs=[pl.BlockSpec((BR, D), lambda i: (i, 0)),
               pl.BlockSpec((BR, D), lambda i: (i, 0))],
    out_shape=[jax.ShapeDtypeStruct((N, D), jnp.float32),
               jax.ShapeDtypeStruct((N, D), jnp.float32)],
)


def _dense2_body(q_ref, a0_ref, wdu_ref, wdv_ref, h_ref, u_ref, v_ref):
    hh = jnp.maximum(q_ref[...] + a0_ref[...], 0.0)
    h_ref[...] = hh
    u_ref[...] = _dot(hh, wdu_ref[...])
    v_ref[...] = _dot(hh, wdv_ref[...])


_dense2 = pl.pallas_call(
    _dense2_body,
    grid=(NB,),
    in_specs=[
        pl.BlockSpec((BR, D), lambda i: (i, 0)),
        pl.BlockSpec((BR, D), lambda i: (i, 0)),
        pl.BlockSpec((D, 1), lambda i: (0, 0)),
        pl.BlockSpec((D, 1), lambda i: (0, 0)),
    ],
    out_specs=[pl.BlockSpec((BR, D), lambda i: (i, 0)),
               pl.BlockSpec((BR, 1), lambda i: (i, 0)),
               pl.BlockSpec((BR, 1), lambda i: (i, 0))],
    out_shape=[jax.ShapeDtypeStruct((N, D), jnp.float32),
               jax.ShapeDtypeStruct((N, 1), jnp.float32),
               jax.ShapeDtypeStruct((N, 1), jnp.float32)],
)


def _edge_bce_body(lg_ref, t_ref, o_ref):
    x = lg_ref[...]
    p = 1.0 / (1.0 + jnp.exp(-x))
    p = jnp.clip(p, 1e-7, 1.0 - 1e-7)
    t = t_ref[...]
    s = jnp.sum(t * jnp.log(p) + (1.0 - t) * jnp.log(1.0 - p))
    o_ref[...] = s.reshape(1, 1)


_edge_bce = pl.pallas_call(
    _edge_bce_body,
    grid=(1,),
    in_specs=[pl.BlockSpec((EROWS, D), lambda i: (0, 0)),
              pl.BlockSpec((EROWS, D), lambda i: (0, 0))],
    out_specs=pl.BlockSpec((1, 1), lambda i: (0, 0)),
    out_shape=jax.ShapeDtypeStruct((1, 1), jnp.float32),
)


def _ynode_body(c_ref, t_ref, y_ref, o_ref):
    cnt = jnp.sum(c_ref[...], axis=0)
    yv = jnp.where(cnt > 0.0, 1.0, 0.0)
    y_ref[...] = yv.reshape(1, N)
    p = jnp.clip(yv, 1e-7, 1.0 - 1e-7)
    t = t_ref[...].reshape(N)
    s = jnp.sum(t * jnp.log(p) + (1.0 - t) * jnp.log(1.0 - p))
    o_ref[...] = s.reshape(1, 1)


_ynode = pl.pallas_call(
    _ynode_body,
    grid=(1,),
    in_specs=[pl.BlockSpec((NW, N), lambda i: (0, 0)),
              pl.BlockSpec((1, N), lambda i: (0, 0))],
    out_specs=[pl.BlockSpec((1, N), lambda i: (0, 0)),
               pl.BlockSpec((1, 1), lambda i: (0, 0))],
    out_shape=[jax.ShapeDtypeStruct((1, N), jnp.float32),
               jax.ShapeDtypeStruct((1, 1), jnp.float32)],
)


_MESH = plsc.VectorSubcoreMesh(core_axis_name="c", subcore_axis_name="s")

_SC_PARAMS = pltpu.CompilerParams()
if "needs_layout_passes" in pltpu.CompilerParams.__dataclass_fields__:
    _SC_PARAMS = dataclasses.replace(_SC_PARAMS, needs_layout_passes=False)


RING = 2
IPF = 4


@functools.partial(
    pl.kernel,
    out_type=jax.ShapeDtypeStruct((NC, NHALF, D), jnp.float32),
    mesh=_MESH,
    scratch_types=[
        pltpu.VMEM((NCH, CH), jnp.int32),
        pltpu.VMEM((NCH, CH), jnp.int32),
        pltpu.VMEM((RING, CH, D), jnp.float32),
        pltpu.VMEM((8, D), jnp.float32),
        pltpu.VMEM_SHARED((NACC, D), jnp.float32),
        pltpu.SemaphoreType.DMA,
        pltpu.SemaphoreType.DMA((RING,)),
        pltpu.SemaphoreType.DMA((IPF,)),
    ],
    compiler_params=_SC_PARAMS,
)
def _sc_agg(m_hbm, src_hbm, dst_hbm, out_hbm, sidx, didx, rows, zbuf, acc,
            sem, gsem, isem):
    core = lax.axis_index("c")
    sid = lax.axis_index("s")
    nbase = core * NHALF

    zv = jnp.zeros((16,), jnp.float32)

    @pl.loop(0, 8)
    def _(r):
        for jj in range(D // 16):
            zbuf[r, pl.ds(jj * 16, 16)] = zv

    @pl.loop(0, NPSH // 8)
    def _(k):
        pltpu.sync_copy(zbuf, acc.at[pl.ds(sid * NPSH + k * 8, 8)])

    plsc.subcore_barrier()

    def fetch_idx(ci, isl):
        base = sid * EPT + ci * CH
        pltpu.async_copy(src_hbm.at[pl.ds(base, CH)], sidx.at[ci],
                         isem.at[isl])
        pltpu.async_copy(dst_hbm.at[pl.ds(base, CH)], didx.at[ci],
                         isem.at[isl])

    def wait_remap(ci, isl):
        base = sid * EPT + ci * CH
        pltpu.make_async_copy(src_hbm.at[pl.ds(base, CH)], sidx.at[ci],
                              isem.at[isl]).wait()
        pltpu.make_async_copy(dst_hbm.at[pl.ds(base, CH)], didx.at[ci],
                              isem.at[isl]).wait()
        for jj in range(CH // 16):
            sl = pl.ds(jj * 16, 16)
            d = didx[ci, sl] - nbase
            ok = (d >= 0) & (d < NHALF)
            didx[ci, sl] = jnp.where(ok, d, NHALF)

    @pl.loop(0, IPF)
    def _(k):
        fetch_idx(k, lax.rem(k, IPF))

    @pl.loop(0, RING)
    def _(r):
        wait_remap(r, lax.rem(r, IPF))
        pltpu.async_copy(m_hbm.at[sidx.at[r]], rows.at[r], gsem.at[r])

    @pl.loop(0, NCH)
    def _(ci):
        r = lax.rem(ci, RING)
        pltpu.make_async_copy(m_hbm.at[sidx.at[ci]], rows.at[r],
                              gsem.at[r]).wait()
        pltpu.sync_copy(rows.at[r], acc.at[didx.at[ci]], add=True)

        @pl.when(ci + IPF < NCH)
        def _():
            fetch_idx(ci + IPF, lax.rem(ci + IPF, IPF))

        @pl.when(ci + RING < NCH)
        def _():
            wait_remap(ci + RING, lax.rem(ci + RING, IPF))
            pltpu.async_copy(m_hbm.at[sidx.at[ci + RING]], rows.at[r],
                             gsem.at[r])

    plsc.subcore_barrier()
    pltpu.sync_copy(acc.at[pl.ds(sid * NPSH, NPSH)],
                    out_hbm.at[core].at[pl.ds(sid * NPSH, NPSH)])


@functools.partial(
    pl.kernel,
    out_type=[jax.ShapeDtypeStruct((E,), jnp.float32),
              jax.ShapeDtypeStruct((NW * N,), jnp.float32)],
    mesh=_MESH,
    scratch_types=[
        pltpu.VMEM((N,), jnp.float32),
        pltpu.VMEM((N,), jnp.float32),
        pltpu.VMEM((EPW,), jnp.int32),
        pltpu.VMEM((EPW,), jnp.int32),
        pltpu.VMEM((EPW,), jnp.float32),
        pltpu.VMEM((N,), jnp.float32),
        pltpu.VMEM((16,), jnp.float32),
        pltpu.VMEM((16,), jnp.float32),
    ],
    compiler_params=_SC_PARAMS,
)
def _sc_edges(u_hbm, v_hbm, src_hbm, dst_hbm, thr_hbm, b_hbm, lg_hbm,
              cnt_hbm, uv, vv, sidx, didx, lgv, cntv, thrv, bvv):
    core = lax.axis_index("c")
    sid = lax.axis_index("s")
    wid = core * NS + sid
    base = wid * EPW

    pltpu.sync_copy(u_hbm, uv)
    pltpu.sync_copy(v_hbm, vv)
    pltpu.sync_copy(src_hbm.at[pl.ds(base, EPW)], sidx)
    pltpu.sync_copy(dst_hbm.at[pl.ds(base, EPW)], didx)
    pltpu.sync_copy(thr_hbm, thrv)
    pltpu.sync_copy(b_hbm, bvv)

    zv = jnp.zeros((16,), jnp.float32)
    ones = jnp.ones((16,), jnp.float32)

    @pl.loop(0, N // 16)
    def _(i):
        cntv[pl.ds(i * 16, 16)] = zv

    thr = thrv[...]
    bv = bvv[...]

    @pl.loop(0, EPW // 16)
    def _(i):
        sl = pl.ds(i * 16, 16)
        si = sidx[sl]
        di = didx[sl]
        uu = plsc.load_gather(uv, [si])
        vg = plsc.load_gather(vv, [di])
        lg = uu + vg + bv
        lgv[sl] = lg
        ind = jnp.where(lg >= thr, ones, zv)
        plsc.addupdate_scatter(cntv, [si], ind)
        plsc.addupdate_scatter(cntv, [di], ind)

    pltpu.sync_copy(lgv, lg_hbm.at[pl.ds(base, EPW)])
    pltpu.sync_copy(cntv, cnt_hbm.at[pl.ds(wid * N, N)])


def _sigmoid_threshold():
    lo = jnp.float32(1.0)
    hi = jnp.float32(2.0)

    def body(_, lh):
        lo_, hi_ = lh
        lob = lax.bitcast_convert_type(lo_, jnp.uint32)
        hib = lax.bitcast_convert_type(hi_, jnp.uint32)
        mid = lax.bitcast_convert_type((lob + hib) // 2, jnp.float32)
        up = jax.nn.sigmoid(mid) >= jnp.float32(0.8)
        return (jnp.where(up, lo_, mid), jnp.where(up, mid, hi_))

    lo, hi = lax.fori_loop(0, 32, body, (lo, hi))
    return hi


def kernel(pos, s, pi, pi_h, reach_h, edge_index, W_enc, b_enc, W1, W2, b_p,
           W_dec, b_dec):
    f32 = jnp.float32
    src = edge_index[0].astype(jnp.int32)
    dst = edge_index[1].astype(jnp.int32)

    pos_c = pos.astype(f32).reshape(N, 1)
    be = b_enc.astype(f32).reshape(1, D)
    bp = b_p.astype(f32).reshape(1, D)
    wdu = W_dec[:D, :].astype(f32).reshape(D, 1)
    wdv = W_dec[D:, :].astype(f32).reshape(D, 1)
    W_enc = W_enc.astype(f32)
    W1 = W1.astype(f32)
    W2 = W2.astype(f32)

    thr = jnp.full((16,), _sigmoid_threshold(), f32)
    bsp = jnp.full((16,), b_dec[0].astype(f32), f32)

    edge_t = [pi_h[i].astype(f32).reshape(EROWS, D) for i in range(1, 5)]
    edge_t.append(pi.astype(f32).reshape(EROWS, D))
    node_t = [reach_h[i].astype(f32).reshape(1, N) for i in range(1, 5)]
    node_t.append(reach_h[4].astype(f32).reshape(1, N))

    h0 = jnp.zeros((N, D), f32)
    ycol0 = s.astype(f32).reshape(N, 1)
    h, ycol = h0, ycol0
    edge_sums = []
    node_sums = []
    for it in range(5):
        inp = jnp.concatenate([pos_c, ycol], axis=1)
        m, q = _dense1(inp, h, W_enc, be, W1, W2, bp)
        aggp = _sc_agg(m, src, dst)
        h, u, v = _dense2(q, aggp.reshape(NPAD, D)[:N], wdu, wdv)
        lg, cnt = _sc_edges(u.reshape(N), v.reshape(N), src, dst, thr, bsp)
        y_row, ysum = _ynode(cnt.reshape(NW, N), node_t[it])
        esum = _edge_bce(lg.reshape(EROWS, D), edge_t[it])
        edge_sums.append(esum[0, 0])
        node_sums.append(ysum[0, 0])
        ycol = y_row.reshape(N, 1)
    ycol_f = ycol

    loss_x = -(edge_sums[4] / E)
    loss_h = 0.0
    for it in range(4):
        loss_h = loss_h + (-(edge_sums[it] / E))
    yloss_x = -(node_sums[4] / N)
    yloss_h = 0.0
    for it in range(4):
        yloss_h = yloss_h + (-(node_sums[it] / N))

    y = ycol_f.reshape(N)
    return (y, loss_x, loss_h, yloss_x, yloss_h)
